# Initial kernel scaffold; baseline (speedup 1.0000x reference)
import jax
import jax.numpy as jnp
from jax.experimental import pallas as pl


def kernel(y_pred, y_ori):
    raise NotImplementedError("write your pallas kernel here")


# speedup vs baseline: 1.6999x; 1.6999x over previous
import functools

import jax
import jax.numpy as jnp
from jax import lax
from jax.experimental import pallas as pl
from jax.experimental.pallas import tpu as pltpu
from jax.experimental.pallas import tpu_sc as plsc

B = 1024
PRED_LEN = 8192
TRUE_LEN = 2048
SF = PRED_LEN // TRUE_LEN
NC = 2
NS = 16
L = 16
NW = NC * NS
ROWS_PER_W = B // NW
T_VALID = TRUE_LEN - 2
NCHUNK = (T_VALID + L - 1) // L


def _sc_body(pred_hbm, ori_hbm, out_hbm, pred_v, ori_v, part_v, sem):
    wid = lax.axis_index("s") * NC + lax.axis_index("c")
    iota = lax.iota(jnp.int32, L)
    ori_v[pl.ds(TRUE_LEN, L)] = jnp.zeros((L,), jnp.float32)

    def row_body(r, carry):
        tot, val = carry
        row = wid * ROWS_PER_W + r
        pltpu.sync_copy(pred_hbm.at[row], pred_v)
        pltpu.sync_copy(ori_hbm.at[row], ori_v.at[pl.ds(0, TRUE_LEN)])

        def chunk_body(j, c):
            asum, acnt = c
            t0 = 1 + j * L
            o_prev = ori_v[pl.ds(t0 - 1, L)]
            o_cur = ori_v[pl.ds(t0, L)]
            o_next = ori_v[pl.ds(t0 + 1, L)]
            t = t0 + iota
            lane_ok = t <= T_VALID
            d1 = o_cur - o_prev
            d2 = o_next - o_cur
            mask = (d1 * d2 < 0.0) & (d1 > 0.0) & lane_ok
            idx = jnp.minimum(t, T_VALID) * SF
            p0 = plsc.load_gather(pred_v, [idx])
            p1 = plsc.load_gather(pred_v, [idx + 1])
            p2 = plsc.load_gather(pred_v, [idx + 2])
            p3 = plsc.load_gather(pred_v, [idx + 3])
            pm = jnp.maximum(jnp.maximum(p0, p1), jnp.maximum(p2, p3))
            d = pm - o_cur
            sq = d * d
            asum = asum + jnp.where(mask, sq, 0.0)
            acnt = acnt + jnp.where(mask, 1.0, 0.0)
            return asum, acnt

        zero = jnp.zeros((L,), jnp.float32)
        asum, acnt = lax.fori_loop(0, NCHUNK, chunk_body, (zero, zero))
        s_v = jnp.full((L,), jnp.sum(asum), jnp.float32)
        c_v = jnp.full((L,), jnp.sum(acnt), jnp.float32)
        bl_v = s_v / jnp.maximum(c_v, 1.0)
        v_v = jnp.where(c_v > 0.0, 1.0, 0.0).astype(jnp.float32)
        return tot + bl_v * v_v, val + v_v

    zero = jnp.zeros((L,), jnp.float32)
    tot, val = lax.fori_loop(0, ROWS_PER_W, row_body, (zero, zero))
    part_v[...] = jnp.where(
        iota == 0, tot, jnp.where(iota == 1, val, jnp.float32(0.0))
    )
    pltpu.sync_copy(part_v, out_hbm.at[wid])


def kernel(y_pred, y_ori):
    mesh = plsc.VectorSubcoreMesh(core_axis_name="c", subcore_axis_name="s")
    run = functools.partial(
        pl.kernel,
        mesh=mesh,
        compiler_params=pltpu.CompilerParams(needs_layout_passes=False),
        out_type=jax.ShapeDtypeStruct((NW, L), jnp.float32),
        scratch_types=[
            pltpu.VMEM((PRED_LEN,), jnp.float32),
            pltpu.VMEM((TRUE_LEN + L,), jnp.float32),
            pltpu.VMEM((L,), jnp.float32),
            pltpu.SemaphoreType.DMA,
        ],
    )(_sc_body)
    parts = run(y_pred, y_ori)
    tot = jnp.sum(parts[:, 0])
    val = jnp.sum(parts[:, 1])
    return tot / jnp.maximum(val, 1.0)


# speedup vs baseline: 3.2644x; 1.9204x over previous
import functools

import jax
import jax.numpy as jnp
from jax import lax
from jax.experimental import pallas as pl
from jax.experimental.pallas import tpu as pltpu
from jax.experimental.pallas import tpu_sc as plsc

B = 1024
PRED_LEN = 8192
TRUE_LEN = 2048
SF = PRED_LEN // TRUE_LEN
NC = 2
NS = 16
L = 16
NW = NC * NS
ROWS_PER_W = B // NW
T_VALID = TRUE_LEN - 2
NFULL = T_VALID // L
TAIL = T_VALID - NFULL * L


def _sc_body(pred_hbm, ori_hbm, out_hbm, pred_v0, pred_v1, ori_v0, ori_v1,
             part_v, sp0, so0, sp1, sp1b):
    wid = lax.axis_index("s") * NC + lax.axis_index("c")
    iota = lax.iota(jnp.int32, L)
    row_base = wid * ROWS_PER_W
    row_last = row_base + ROWS_PER_W - 1
    sem_p = (sp0, sp1)
    sem_o = (so0, sp1b)
    pred_bufs = (pred_v0, pred_v1)
    ori_bufs = (ori_v0, ori_v1)

    for b in range(2):
        ori_bufs[b][pl.ds(TRUE_LEN, L)] = jnp.zeros((L,), jnp.float32)

    def start(row, b):
        pltpu.async_copy(pred_hbm.at[row], pred_bufs[b], sem_p[b])
        pltpu.async_copy(ori_hbm.at[row], ori_bufs[b].at[pl.ds(0, TRUE_LEN)],
                         sem_o[b])

    def wait(b):
        pltpu.make_async_copy(pred_hbm.at[0], pred_bufs[b], sem_p[b]).wait()
        pltpu.make_async_copy(ori_hbm.at[0],
                              ori_bufs[b].at[pl.ds(0, TRUE_LEN)],
                              sem_o[b]).wait()

    def row_loss(b):
        pv = pred_bufs[b]
        ov = ori_bufs[b]

        def chunk_body(j, c):
            asum, acnt, idx = c
            t0 = 1 + j * L
            o_prev = ov[pl.ds(t0 - 1, L)]
            o_cur = ov[pl.ds(t0, L)]
            o_next = ov[pl.ds(t0 + 1, L)]
            d1 = o_cur - o_prev
            d2 = o_next - o_cur
            mask = (d1 * d2 < 0.0) & (d1 > 0.0)
            p0 = plsc.load_gather(pv, [idx])
            p1 = plsc.load_gather(pv, [idx + 1])
            p2 = plsc.load_gather(pv, [idx + 2])
            p3 = plsc.load_gather(pv, [idx + 3])
            pm = jnp.maximum(jnp.maximum(p0, p1), jnp.maximum(p2, p3))
            d = pm - o_cur
            sq = d * d
            asum = asum + jnp.where(mask, sq, 0.0)
            acnt = acnt + jnp.where(mask, 1.0, 0.0)
            return asum, acnt, idx + (SF * L)

        zero = jnp.zeros((L,), jnp.float32)
        idx0 = (iota + 1) * SF
        asum, acnt, _ = lax.fori_loop(
            0, NFULL, chunk_body, (zero, zero, idx0), unroll=2
        )

        t0 = 1 + NFULL * L
        t = t0 + iota
        lane_ok = iota < TAIL
        o_prev = ov[pl.ds(t0 - 1, L)]
        o_cur = ov[pl.ds(t0, L)]
        o_next = ov[pl.ds(t0 + 1, L)]
        d1 = o_cur - o_prev
        d2 = o_next - o_cur
        mask = (d1 * d2 < 0.0) & (d1 > 0.0) & lane_ok
        idx = jnp.minimum(t, T_VALID) * SF
        p0 = plsc.load_gather(pv, [idx])
        p1 = plsc.load_gather(pv, [idx + 1])
        p2 = plsc.load_gather(pv, [idx + 2])
        p3 = plsc.load_gather(pv, [idx + 3])
        pm = jnp.maximum(jnp.maximum(p0, p1), jnp.maximum(p2, p3))
        d = pm - o_cur
        sq = d * d
        asum = asum + jnp.where(mask, sq, 0.0)
        acnt = acnt + jnp.where(mask, 1.0, 0.0)

        s_v = jnp.full((L,), jnp.sum(asum), jnp.float32)
        c_v = jnp.full((L,), jnp.sum(acnt), jnp.float32)
        bl_v = s_v / jnp.maximum(c_v, 1.0)
        v_v = jnp.where(c_v > 0.0, 1.0, 0.0).astype(jnp.float32)
        return bl_v * v_v, v_v

    start(row_base, 0)
    start(row_base + 1, 1)

    def pair_body(g, carry):
        tot, val = carry
        r0 = row_base + 2 * g
        wait(0)
        bl0, v0 = row_loss(0)
        start(jnp.minimum(r0 + 2, row_last), 0)
        wait(1)
        bl1, v1 = row_loss(1)
        start(jnp.minimum(r0 + 3, row_last), 1)
        return tot + bl0 + bl1, val + v0 + v1

    zero = jnp.zeros((L,), jnp.float32)
    tot, val = lax.fori_loop(0, ROWS_PER_W // 2, pair_body, (zero, zero))
    wait(0)
    wait(1)

    part_v[...] = jnp.where(
        iota == 0, tot, jnp.where(iota == 1, val, jnp.float32(0.0))
    )
    pltpu.sync_copy(part_v, out_hbm.at[wid])


def kernel(y_pred, y_ori):
    mesh = plsc.VectorSubcoreMesh(core_axis_name="c", subcore_axis_name="s")
    run = functools.partial(
        pl.kernel,
        mesh=mesh,
        compiler_params=pltpu.CompilerParams(needs_layout_passes=False),
        out_type=jax.ShapeDtypeStruct((NW, L), jnp.float32),
        scratch_types=[
            pltpu.VMEM((PRED_LEN,), jnp.float32),
            pltpu.VMEM((PRED_LEN,), jnp.float32),
            pltpu.VMEM((TRUE_LEN + L,), jnp.float32),
            pltpu.VMEM((TRUE_LEN + L,), jnp.float32),
            pltpu.VMEM((L,), jnp.float32),
            pltpu.SemaphoreType.DMA,
            pltpu.SemaphoreType.DMA,
            pltpu.SemaphoreType.DMA,
            pltpu.SemaphoreType.DMA,
        ],
    )(_sc_body)
    parts = run(y_pred, y_ori)
    tot = jnp.sum(parts[:, 0])
    val = jnp.sum(parts[:, 1])
    return tot / jnp.maximum(val, 1.0)
